# Initial kernel scaffold; baseline (speedup 1.0000x reference)
import functools

import jax
import jax.numpy as jnp
from jax import lax
from jax.experimental import pallas as pl
from jax.experimental.pallas import tpu as pltpu
from jax.experimental.pallas import tpu_sc as plsc

_N = 10000
_D = 128
_E = 320000
_L = 16

_NC = 2
_NS = 16
_NW = _NC * _NS
_EPT = _E // _NW
_CH = 80
_NCH = _EPT // _CH
_RPT = _N // _NS
_ZR = 125


def _seg_sum_partials(h, edge_index):
  mesh = plsc.VectorSubcoreMesh(core_axis_name="c", subcore_axis_name="s")

  @functools.partial(
      pl.kernel,
      mesh=mesh,
      out_type=jax.ShapeDtypeStruct((_NC, _N, _D), jnp.float32),
      scratch_types=[
          pltpu.VMEM((2, _CH), jnp.int32),
          pltpu.VMEM((_CH, _D), jnp.float32),
          pltpu.VMEM((_ZR, _D), jnp.float32),
          pltpu.VMEM_SHARED((_N, _D), jnp.float32),
          pltpu.SemaphoreType.DMA,
      ],
  )
  def seg_kernel(h_hbm, ei_hbm, out_hbm, idx_v, rows_v, zero_v, acc_sh, sem):
    c = lax.axis_index("c")
    s = lax.axis_index("s")
    wid = s * _NC + c

    def _zrow(i, carry):
      for k in range(_D // 16):
        zero_v[i, pl.ds(k * 16, 16)] = jnp.zeros((16,), jnp.float32)
      return carry
    lax.fori_loop(0, _ZR, _zrow, 0)
    for r in range(_RPT // _ZR):
      pltpu.sync_copy(zero_v, acc_sh.at[pl.ds(s * _RPT + r * _ZR, _ZR)])
    plsc.subcore_barrier()

    base = wid * _EPT

    def _chunk(j, carry):
      off = base + j * _CH
      pltpu.sync_copy(ei_hbm.at[:, pl.ds(off, _CH)], idx_v)
      pltpu.async_copy(h_hbm.at[idx_v.at[0]], rows_v, sem).wait()
      pltpu.sync_copy(rows_v, acc_sh.at[idx_v.at[1]], add=True)
      return carry
    lax.fori_loop(0, _NCH, _chunk, 0)
    plsc.subcore_barrier()

    pltpu.sync_copy(acc_sh.at[pl.ds(s * _RPT, _RPT)],
                    out_hbm.at[c, pl.ds(s * _RPT, _RPT)])

  return seg_kernel(h, edge_index)


_RB = 1000


def _mlp1(x, agg, Wa, ba, Wb, bb):
  def body(x_ref, a_ref, wa_ref, ba_ref, wb_ref, bb_ref, o_ref):
    m = x_ref[...] + a_ref[0] + a_ref[1]
    m = jnp.maximum(
        jnp.dot(m, wa_ref[...], preferred_element_type=jnp.float32)
        + ba_ref[...], 0.0)
    hh = (jnp.dot(m, wb_ref[...], preferred_element_type=jnp.float32)
          + bb_ref[...])
    o_ref[...] = jnp.where(hh > 0, hh, jnp.expm1(hh))

  return pl.pallas_call(
      body,
      grid=(_N // _RB,),
      in_specs=[
          pl.BlockSpec((_RB, _D), lambda i: (i, 0)),
          pl.BlockSpec((2, _RB, _D), lambda i: (0, i, 0)),
          pl.BlockSpec((_D, _D), lambda i: (0, 0)),
          pl.BlockSpec((1, _D), lambda i: (0, 0)),
          pl.BlockSpec((_D, _D), lambda i: (0, 0)),
          pl.BlockSpec((1, _D), lambda i: (0, 0)),
      ],
      out_specs=pl.BlockSpec((_RB, _D), lambda i: (i, 0)),
      out_shape=jax.ShapeDtypeStruct((_N, _D), jnp.float32),
  )(x, agg, Wa, ba.reshape(1, _D), Wb, bb.reshape(1, _D))


def _mlp2(h, agg, Wa, ba, Wb, bb, Wf, bf):
  def body(h_ref, a_ref, wa_ref, ba_ref, wb_ref, bb_ref, wf_ref, bf_ref,
           o_ref):
    m = h_ref[...] + a_ref[0] + a_ref[1]
    m = jnp.maximum(
        jnp.dot(m, wa_ref[...], preferred_element_type=jnp.float32)
        + ba_ref[...], 0.0)
    h2 = (jnp.dot(m, wb_ref[...], preferred_element_type=jnp.float32)
          + bb_ref[...])
    h2 = jnp.where(h2 > 0, h2, jnp.expm1(h2))
    z = (jnp.dot(h2, wf_ref[...], preferred_element_type=jnp.float32)
         + bf_ref[...])
    o_ref[...] = 1.0 / (1.0 + jnp.exp(-z))

  return pl.pallas_call(
      body,
      grid=(_N // _RB,),
      in_specs=[
          pl.BlockSpec((_RB, _D), lambda i: (i, 0)),
          pl.BlockSpec((2, _RB, _D), lambda i: (0, i, 0)),
          pl.BlockSpec((_D, _D), lambda i: (0, 0)),
          pl.BlockSpec((1, _D), lambda i: (0, 0)),
          pl.BlockSpec((_D, _D), lambda i: (0, 0)),
          pl.BlockSpec((1, _D), lambda i: (0, 0)),
          pl.BlockSpec((_D, _L), lambda i: (0, 0)),
          pl.BlockSpec((1, _L), lambda i: (0, 0)),
      ],
      out_specs=pl.BlockSpec((_RB, _L), lambda i: (i, 0)),
      out_shape=jax.ShapeDtypeStruct((_N, _L), jnp.float32),
  )(h, agg, Wa, ba.reshape(1, _D), Wb, bb.reshape(1, _D), Wf,
    bf.reshape(1, _L))


def kernel(x, edge_index, W11, b11, W12, b12, W21, b21, W22, b22, Wf, bf):
  agg1 = _seg_sum_partials(x, edge_index)
  h1 = _mlp1(x, agg1, W11, b11, W12, b12)
  agg2 = _seg_sum_partials(h1, edge_index)
  return _mlp2(h1, agg2, W21, b21, W22, b22, Wf, bf)


# speedup vs baseline: 4.6805x; 4.6805x over previous
import functools

import jax
import jax.numpy as jnp
from jax import lax
from jax.experimental import pallas as pl
from jax.experimental.pallas import tpu as pltpu
from jax.experimental.pallas import tpu_sc as plsc

_N = 10000
_D = 128
_E = 320000
_L = 16

_NC = 2
_NS = 16
_NW = _NC * _NS
_EPT = _E // _NW
_CH = 80
_NCH = _EPT // _CH
_NP = 10112
_RPT = _NP // _NS


def _seg_sum_partials(h, src, dst):
  mesh = plsc.VectorSubcoreMesh(core_axis_name="c", subcore_axis_name="s")

  @functools.partial(
      pl.kernel,
      mesh=mesh,
      out_type=jax.ShapeDtypeStruct((_NC, _NP, _D), jnp.float32),
      scratch_types=[
          pltpu.VMEM((2, _CH), jnp.int32),
          pltpu.VMEM((_CH, _D), jnp.float32),
          pltpu.VMEM_SHARED((_NP, _D), jnp.float32),
          pltpu.SemaphoreType.DMA,
      ],
  )
  def seg_kernel(h_hbm, src_hbm, dst_hbm, out_hbm, idx_v, rows_v,
                 acc_sh, sem):
    c = lax.axis_index("c")
    s = lax.axis_index("s")
    wid = s * _NC + c

    def _zrow(i, carry):
      for k in range(_D // 16):
        rows_v[i, pl.ds(k * 16, 16)] = jnp.zeros((16,), jnp.float32)
      return carry
    lax.fori_loop(0, _CH, _zrow, 0)
    for r in range(_RPT // _CH):
      pltpu.sync_copy(rows_v, acc_sh.at[pl.ds(s * _RPT + r * _CH, _CH)])
    _REM = _RPT - (_RPT // _CH) * _CH
    pltpu.sync_copy(
        rows_v.at[pl.ds(0, _REM)],
        acc_sh.at[pl.ds(s * _RPT + (_RPT // _CH) * _CH, _REM)])
    plsc.subcore_barrier()

    base = wid * _EPT

    def _chunk(j, carry):
      off = base + j * _CH
      pltpu.sync_copy(src_hbm.at[pl.ds(off, _CH)], idx_v.at[0])
      pltpu.sync_copy(dst_hbm.at[pl.ds(off, _CH)], idx_v.at[1])
      pltpu.async_copy(h_hbm.at[idx_v.at[0]], rows_v, sem).wait()
      pltpu.sync_copy(rows_v, acc_sh.at[idx_v.at[1]], add=True)
      return carry
    lax.fori_loop(0, _NCH, _chunk, 0)
    plsc.subcore_barrier()

    pltpu.sync_copy(acc_sh.at[pl.ds(s * _RPT, _RPT)],
                    out_hbm.at[c, pl.ds(s * _RPT, _RPT)])

  return seg_kernel(h, src, dst)


_RB = 1000


def _mlp1(x, agg, Wa, ba, Wb, bb):
  def body(x_ref, a_ref, wa_ref, ba_ref, wb_ref, bb_ref, o_ref):
    m = x_ref[...] + a_ref[0] + a_ref[1]
    m = jnp.maximum(
        jnp.dot(m, wa_ref[...], preferred_element_type=jnp.float32)
        + ba_ref[...], 0.0)
    hh = (jnp.dot(m, wb_ref[...], preferred_element_type=jnp.float32)
          + bb_ref[...])
    o_ref[...] = jnp.where(hh > 0, hh, jnp.exp(jnp.minimum(hh, 0.0)) - 1.0)

  return pl.pallas_call(
      body,
      grid=(_N // _RB,),
      in_specs=[
          pl.BlockSpec((_RB, _D), lambda i: (i, 0)),
          pl.BlockSpec((2, _RB, _D), lambda i: (0, i, 0)),
          pl.BlockSpec((_D, _D), lambda i: (0, 0)),
          pl.BlockSpec((1, _D), lambda i: (0, 0)),
          pl.BlockSpec((_D, _D), lambda i: (0, 0)),
          pl.BlockSpec((1, _D), lambda i: (0, 0)),
      ],
      out_specs=pl.BlockSpec((_RB, _D), lambda i: (i, 0)),
      out_shape=jax.ShapeDtypeStruct((_N, _D), jnp.float32),
  )(x, agg, Wa, ba.reshape(1, _D), Wb, bb.reshape(1, _D))


def _mlp2(h, agg, Wa, ba, Wb, bb, Wf, bf):
  def body(h_ref, a_ref, wa_ref, ba_ref, wb_ref, bb_ref, wf_ref, bf_ref,
           o_ref):
    m = h_ref[...] + a_ref[0] + a_ref[1]
    m = jnp.maximum(
        jnp.dot(m, wa_ref[...], preferred_element_type=jnp.float32)
        + ba_ref[...], 0.0)
    h2 = (jnp.dot(m, wb_ref[...], preferred_element_type=jnp.float32)
          + bb_ref[...])
    h2 = jnp.where(h2 > 0, h2, jnp.exp(jnp.minimum(h2, 0.0)) - 1.0)
    z = (jnp.dot(h2, wf_ref[...], preferred_element_type=jnp.float32)
         + bf_ref[...])
    o_ref[...] = 1.0 / (1.0 + jnp.exp(-z))

  return pl.pallas_call(
      body,
      grid=(_N // _RB,),
      in_specs=[
          pl.BlockSpec((_RB, _D), lambda i: (i, 0)),
          pl.BlockSpec((2, _RB, _D), lambda i: (0, i, 0)),
          pl.BlockSpec((_D, _D), lambda i: (0, 0)),
          pl.BlockSpec((1, _D), lambda i: (0, 0)),
          pl.BlockSpec((_D, _D), lambda i: (0, 0)),
          pl.BlockSpec((1, _D), lambda i: (0, 0)),
          pl.BlockSpec((_D, _L), lambda i: (0, 0)),
          pl.BlockSpec((1, _L), lambda i: (0, 0)),
      ],
      out_specs=pl.BlockSpec((_RB, _L), lambda i: (i, 0)),
      out_shape=jax.ShapeDtypeStruct((_N, _L), jnp.float32),
  )(h, agg, Wa, ba.reshape(1, _D), Wb, bb.reshape(1, _D), Wf,
    bf.reshape(1, _L))


def kernel(x, edge_index, W11, b11, W12, b12, W21, b21, W22, b22, Wf, bf):
  src = edge_index[0]
  dst = edge_index[1]
  agg1 = _seg_sum_partials(x, src, dst)
  h1 = _mlp1(x, agg1, W11, b11, W12, b12)
  agg2 = _seg_sum_partials(h1, src, dst)
  return _mlp2(h1, agg2, W21, b21, W22, b22, Wf, bf)


# speedup vs baseline: 4.7310x; 1.0108x over previous
import functools

import jax
import jax.numpy as jnp
from jax import lax
from jax.experimental import pallas as pl
from jax.experimental.pallas import tpu as pltpu
from jax.experimental.pallas import tpu_sc as plsc

_N = 10000
_D = 128
_E = 320000
_L = 16

_NC = 2
_NS = 16
_NW = _NC * _NS
_CH = 128
_NCH = 79
_EPT = _NCH * _CH
_EP = _NW * _EPT
_NP = 10112
_RPT = _NP // _NS


def _seg_sum_partials(h, src1, dst1):
  mesh = plsc.VectorSubcoreMesh(core_axis_name="c", subcore_axis_name="s")

  @functools.partial(
      pl.kernel,
      mesh=mesh,
      out_type=jax.ShapeDtypeStruct((_NC, _NP, _D), jnp.float32),
      scratch_types=[
          pltpu.VMEM((2, _CH), jnp.int32),
          pltpu.VMEM((2, _CH), jnp.int32),
          pltpu.VMEM((2, _CH, _D), jnp.float32),
          pltpu.VMEM_SHARED((_NP, _D), jnp.float32),
          pltpu.SemaphoreType.DMA,
          pltpu.SemaphoreType.DMA,
          pltpu.SemaphoreType.DMA,
          pltpu.SemaphoreType.DMA,
      ],
  )
  def seg_kernel(h_hbm, src_hbm, dst_hbm, out_hbm, ia_v, ib_v, rows_v,
                 acc_sh, semia, semib, semg0, semg1):
    c = lax.axis_index("c")
    s = lax.axis_index("s")
    wid = s * _NC + c
    base = wid * _EPT

    def idx_load(j, buf, sem):
      off = base + j * _CH
      d1 = pltpu.async_copy(src_hbm.at[pl.ds(off, _CH)], buf.at[0], sem)
      d2 = pltpu.async_copy(dst_hbm.at[pl.ds(off, _CH)], buf.at[1], sem)
      return d1, d2

    def idx_wait(buf, sem):
      pltpu.make_async_copy(src_hbm.at[pl.ds(0, _CH)], buf.at[0], sem).wait()
      pltpu.make_async_copy(src_hbm.at[pl.ds(0, _CH)], buf.at[1], sem).wait()

    def g_start(buf, rb, sem):
      pltpu.async_copy(h_hbm.at[buf.at[0]], rows_v.at[rb], sem)

    def g_wait(buf, rb, sem):
      pltpu.make_async_copy(h_hbm.at[buf.at[0]], rows_v.at[rb], sem).wait()

    def scat(buf, rb):
      pltpu.sync_copy(rows_v.at[rb], acc_sh.at[buf.at[1]], add=True)

    d1, d2 = idx_load(0, ia_v, semia)
    d1.wait()
    d2.wait()
    g_start(ia_v, 0, semg0)
    idx_load(1, ib_v, semib)

    def _zrow(i, carry):
      for k in range(_D // 16):
        rows_v[1, i, pl.ds(k * 16, 16)] = jnp.zeros((16,), jnp.float32)
      return carry
    lax.fori_loop(0, _CH, _zrow, 0)
    for r in range(_RPT // _CH):
      pltpu.sync_copy(rows_v.at[1], acc_sh.at[pl.ds(s * _RPT + r * _CH, _CH)])
    _REM = _RPT - (_RPT // _CH) * _CH
    pltpu.sync_copy(
        rows_v.at[1, pl.ds(0, _REM)],
        acc_sh.at[pl.ds(s * _RPT + (_RPT // _CH) * _CH, _REM)])
    plsc.subcore_barrier()

    def _pair(k, carry):
      j0 = 2 * k
      idx_wait(ib_v, semib)
      g_wait(ia_v, 0, semg0)
      g_start(ib_v, 1, semg1)
      scat(ia_v, 0)
      idx_load(jnp.minimum(j0 + 2, _NCH - 1), ia_v, semia)
      idx_wait(ia_v, semia)
      g_wait(ib_v, 1, semg1)
      g_start(ia_v, 0, semg0)
      scat(ib_v, 1)
      idx_load(jnp.minimum(j0 + 3, _NCH - 1), ib_v, semib)
      return carry
    lax.fori_loop(0, (_NCH - 1) // 2, _pair, 0)
    idx_wait(ib_v, semib)
    g_wait(ia_v, 0, semg0)
    scat(ia_v, 0)
    plsc.subcore_barrier()

    pltpu.sync_copy(acc_sh.at[pl.ds(s * _RPT, _RPT)],
                    out_hbm.at[c, pl.ds(s * _RPT, _RPT)])

  return seg_kernel(h, src1, dst1)


_RB = 1000


def _mlp1(x, agg, Wa, ba, Wb, bb):
  def body(x_ref, a_ref, wa_ref, ba_ref, wb_ref, bb_ref, o_ref):
    m = x_ref[...] + a_ref[0] + a_ref[1]
    m = jnp.maximum(
        jnp.dot(m, wa_ref[...], preferred_element_type=jnp.float32)
        + ba_ref[...], 0.0)
    hh = (jnp.dot(m, wb_ref[...], preferred_element_type=jnp.float32)
          + bb_ref[...])
    o_ref[...] = jnp.where(hh > 0, hh, jnp.exp(jnp.minimum(hh, 0.0)) - 1.0)

  return pl.pallas_call(
      body,
      grid=(_N // _RB,),
      in_specs=[
          pl.BlockSpec((_RB, _D), lambda i: (i, 0)),
          pl.BlockSpec((2, _RB, _D), lambda i: (0, i, 0)),
          pl.BlockSpec((_D, _D), lambda i: (0, 0)),
          pl.BlockSpec((1, _D), lambda i: (0, 0)),
          pl.BlockSpec((_D, _D), lambda i: (0, 0)),
          pl.BlockSpec((1, _D), lambda i: (0, 0)),
      ],
      out_specs=pl.BlockSpec((_RB, _D), lambda i: (i, 0)),
      out_shape=jax.ShapeDtypeStruct((_N, _D), jnp.float32),
  )(x, agg, Wa, ba.reshape(1, _D), Wb, bb.reshape(1, _D))


def _mlp2(h, agg, Wa, ba, Wb, bb, Wf, bf):
  def body(h_ref, a_ref, wa_ref, ba_ref, wb_ref, bb_ref, wf_ref, bf_ref,
           o_ref):
    m = h_ref[...] + a_ref[0] + a_ref[1]
    m = jnp.maximum(
        jnp.dot(m, wa_ref[...], preferred_element_type=jnp.float32)
        + ba_ref[...], 0.0)
    h2 = (jnp.dot(m, wb_ref[...], preferred_element_type=jnp.float32)
          + bb_ref[...])
    h2 = jnp.where(h2 > 0, h2, jnp.exp(jnp.minimum(h2, 0.0)) - 1.0)
    z = (jnp.dot(h2, wf_ref[...], preferred_element_type=jnp.float32)
         + bf_ref[...])
    o_ref[...] = 1.0 / (1.0 + jnp.exp(-z))

  return pl.pallas_call(
      body,
      grid=(_N // _RB,),
      in_specs=[
          pl.BlockSpec((_RB, _D), lambda i: (i, 0)),
          pl.BlockSpec((2, _RB, _D), lambda i: (0, i, 0)),
          pl.BlockSpec((_D, _D), lambda i: (0, 0)),
          pl.BlockSpec((1, _D), lambda i: (0, 0)),
          pl.BlockSpec((_D, _D), lambda i: (0, 0)),
          pl.BlockSpec((1, _D), lambda i: (0, 0)),
          pl.BlockSpec((_D, _L), lambda i: (0, 0)),
          pl.BlockSpec((1, _L), lambda i: (0, 0)),
      ],
      out_specs=pl.BlockSpec((_RB, _L), lambda i: (i, 0)),
      out_shape=jax.ShapeDtypeStruct((_N, _L), jnp.float32),
  )(h, agg, Wa, ba.reshape(1, _D), Wb, bb.reshape(1, _D), Wf,
    bf.reshape(1, _L))


def kernel(x, edge_index, W11, b11, W12, b12, W21, b21, W22, b22, Wf, bf):
  pad = _EP - _E
  src1 = jnp.concatenate([edge_index[0], jnp.zeros((pad,), jnp.int32)])
  dst1 = jnp.concatenate([edge_index[1], jnp.full((pad,), _N, jnp.int32)])
  agg1 = _seg_sum_partials(x, src1, dst1)
  h1 = _mlp1(x, agg1, W11, b11, W12, b12)
  agg2 = _seg_sum_partials(h1, src1, dst1)
  return _mlp2(h1, agg2, W21, b21, W22, b22, Wf, bf)
